# Initial kernel scaffold; baseline (speedup 1.0000x reference)
import jax
import jax.numpy as jnp
from jax.experimental import pallas as pl


def kernel(x, edge_index, W, b):
    raise NotImplementedError("write your pallas kernel here")


# speedup vs baseline: 11.9869x; 11.9869x over previous
import functools

import jax
import jax.numpy as jnp
from jax import lax
from jax.experimental import pallas as pl
from jax.experimental.pallas import tpu as pltpu
from jax.experimental.pallas import tpu_sc as plsc

N = 10000
E = 320000
D = 128
NP = 10112

NC = 2
NS = 16
NW = NC * NS
EPW = E // NW
FPW = D // NW
EB = 4000
NEB = E // EB

_mesh = plsc.VectorSubcoreMesh(
    core_axis_name="c", subcore_axis_name="s", num_cores=NC, num_subcores=NS)
_params = pltpu.CompilerParams(needs_layout_passes=False)


@functools.partial(
    pl.kernel,
    out_type=jax.ShapeDtypeStruct((NW, 1, NP), jnp.float32),
    mesh=_mesh,
    compiler_params=_params,
    scratch_types=[
        pltpu.VMEM((EPW,), jnp.int32),
        pltpu.VMEM((NP,), jnp.float32),
    ],
)
def _deg_kernel(dst1, degp, didx, acc):
    c = lax.axis_index("c")
    s = lax.axis_index("s")
    wid = c * NS + s
    pltpu.sync_copy(dst1.at[pl.ds(wid * EPW, EPW)], didx)
    zero16 = jnp.zeros((16,), jnp.float32)
    ones16 = jnp.ones((16,), jnp.float32)

    def zbody(i, carry):
        acc[pl.ds(i * 16, 16)] = zero16
        return carry

    lax.fori_loop(0, NP // 16, zbody, 0)

    def body(j, carry):
        dv = didx[pl.ds(j * 16, 16)]
        plsc.addupdate_scatter(acc, [dv], ones16)
        return carry

    lax.fori_loop(0, EPW // 16, body, 0)
    pltpu.sync_copy(acc, degp.at[wid, 0])


@functools.partial(
    pl.kernel,
    out_type=jax.ShapeDtypeStruct((NW, FPW, NP), jnp.float32),
    mesh=_mesh,
    compiler_params=_params,
    scratch_types=[
        [pltpu.VMEM((NP,), jnp.float32) for _ in range(FPW)],
        [pltpu.VMEM((NP,), jnp.float32) for _ in range(FPW)],
        [pltpu.VMEM((EB,), jnp.int32) for _ in range(2)],
        [pltpu.VMEM((EB,), jnp.int32) for _ in range(2)],
        [pltpu.SemaphoreType.DMA for _ in range(4)],
    ],
)
def _seg_kernel(h2s, src1, dst1, outp, hs, accs, sbs, dbs, sems):
    c = lax.axis_index("c")
    s = lax.axis_index("s")
    wid = c * NS + s
    for f in range(FPW):
        pltpu.sync_copy(h2s.at[wid, f], hs[f])
    zero16 = jnp.zeros((16,), jnp.float32)

    def zbody(i, carry):
        for f in range(FPW):
            accs[f][pl.ds(i * 16, 16)] = zero16
        return carry

    lax.fori_loop(0, NP // 16, zbody, 0)

    for p in range(2):
        pltpu.async_copy(src1.at[pl.ds(p * EB, EB)], sbs[p], sems[2 * p])
        pltpu.async_copy(dst1.at[pl.ds(p * EB, EB)], dbs[p], sems[2 * p + 1])

    def ebody(k, carry):
        for p in range(2):
            ck = k * 2 + p
            pltpu.make_async_copy(
                src1.at[pl.ds(ck * EB, EB)], sbs[p], sems[2 * p]).wait()
            pltpu.make_async_copy(
                dst1.at[pl.ds(ck * EB, EB)], dbs[p], sems[2 * p + 1]).wait()

            def ibody(i, icarry):
                sv = sbs[p][pl.ds(i * 16, 16)]
                dv = dbs[p][pl.ds(i * 16, 16)]
                for f in range(FPW):
                    g = plsc.load_gather(hs[f], [sv])
                    plsc.addupdate_scatter(accs[f], [dv], g)
                return icarry

            lax.fori_loop(0, EB // 16, ibody, 0)

            @pl.when(ck + 2 < NEB)
            def _():
                pltpu.async_copy(
                    src1.at[pl.ds((ck + 2) * EB, EB)], sbs[p], sems[2 * p])
                pltpu.async_copy(
                    dst1.at[pl.ds((ck + 2) * EB, EB)], dbs[p], sems[2 * p + 1])

        return carry

    lax.fori_loop(0, NEB // 2, ebody, 0)
    for f in range(FPW):
        pltpu.sync_copy(accs[f], outp.at[wid, f])


def _mm_body(x_ref, w_ref, degp_ref, h2s_ref):
    deg = jnp.sum(degp_ref[...][:, 0, :], axis=0)
    dis = lax.rsqrt(jnp.maximum(deg, 1.0))
    h = jnp.dot(x_ref[...], w_ref[...], preferred_element_type=jnp.float32)
    h2 = h * dis[:N, None]
    hT = jnp.concatenate(
        [h2.T, jnp.zeros((D, NP - N), jnp.float32)], axis=1)
    h2s_ref[...] = hT.reshape(NW, FPW, NP)


_mm = pl.pallas_call(
    _mm_body,
    out_shape=jax.ShapeDtypeStruct((NW, FPW, NP), jnp.float32),
)


def _fin_body(outp_ref, degp_ref, b_ref, out_ref):
    deg = jnp.sum(degp_ref[...][:, 0, :], axis=0)
    dis = lax.rsqrt(jnp.maximum(deg, 1.0))
    rows = outp_ref[...].reshape(D, NP)
    out_ref[...] = rows[:, :N].T * dis[:N, None] + b_ref[...]


_fin = pl.pallas_call(
    _fin_body,
    out_shape=jax.ShapeDtypeStruct((N, D), jnp.float32),
)


def kernel(x, edge_index, W, b):
    src1 = edge_index[0]
    dst1 = edge_index[1]
    degp = _deg_kernel(dst1)
    h2s = _mm(x, W, degp)
    outp = _seg_kernel(h2s, src1, dst1)
    return _fin(outp, degp, b.reshape(1, D))


# speedup vs baseline: 26.6935x; 2.2269x over previous
import functools

import jax
import jax.numpy as jnp
from jax import lax
from jax.experimental import pallas as pl
from jax.experimental.pallas import tpu as pltpu
from jax.experimental.pallas import tpu_sc as plsc

N = 10000
E = 320000
D = 128
NP = 10112

NC = 2
NS = 16
NW = NC * NS
EPW = E // NW
FPW = D // NW
EB = 4000
NEB = E // EB

_mesh = plsc.VectorSubcoreMesh(
    core_axis_name="c", subcore_axis_name="s", num_cores=NC, num_subcores=NS)
_params = pltpu.CompilerParams(needs_layout_passes=False)


@functools.partial(
    pl.kernel,
    out_type=jax.ShapeDtypeStruct((NW, 1, NP), jnp.float32),
    mesh=_mesh,
    compiler_params=_params,
    scratch_types=[
        pltpu.VMEM((EPW,), jnp.int32),
        pltpu.VMEM((NP,), jnp.float32),
    ],
)
def _deg_kernel(dst1, degp, didx, acc):
    c = lax.axis_index("c")
    s = lax.axis_index("s")
    wid = c * NS + s
    pltpu.sync_copy(dst1.at[pl.ds(wid * EPW, EPW)], didx)
    zero16 = jnp.zeros((16,), jnp.float32)
    ones16 = jnp.ones((16,), jnp.float32)

    @plsc.parallel_loop(0, NP // 16, unroll=8)
    def _(i):
        acc[pl.ds(i * 16, 16)] = zero16

    @plsc.parallel_loop(0, EPW // 16, unroll=8)
    def _(j):
        dv = didx[pl.ds(j * 16, 16)]
        plsc.addupdate_scatter(acc, [dv], ones16)

    pltpu.sync_copy(acc, degp.at[wid, 0])


@functools.partial(
    pl.kernel,
    out_type=jax.ShapeDtypeStruct((NW, FPW, NP), jnp.float32),
    mesh=_mesh,
    compiler_params=_params,
    scratch_types=[
        [pltpu.VMEM((NP,), jnp.float32) for _ in range(FPW)],
        [pltpu.VMEM((NP,), jnp.float32) for _ in range(FPW)],
        [pltpu.VMEM((EB,), jnp.int32) for _ in range(2)],
        [pltpu.VMEM((EB,), jnp.int32) for _ in range(2)],
        [pltpu.SemaphoreType.DMA for _ in range(4)],
    ],
)
def _seg_kernel(h2s, src1, dst1, outp, hs, accs, sbs, dbs, sems):
    c = lax.axis_index("c")
    s = lax.axis_index("s")
    wid = c * NS + s
    for f in range(FPW):
        pltpu.sync_copy(h2s.at[wid, f], hs[f])
    zero16 = jnp.zeros((16,), jnp.float32)

    @plsc.parallel_loop(0, NP // 16, unroll=4)
    def _(i):
        for f in range(FPW):
            accs[f][pl.ds(i * 16, 16)] = zero16

    for p in range(2):
        pltpu.async_copy(src1.at[pl.ds(p * EB, EB)], sbs[p], sems[2 * p])
        pltpu.async_copy(dst1.at[pl.ds(p * EB, EB)], dbs[p], sems[2 * p + 1])

    def ebody(k, carry):
        for p in range(2):
            ck = k * 2 + p
            pltpu.make_async_copy(
                src1.at[pl.ds(ck * EB, EB)], sbs[p], sems[2 * p]).wait()
            pltpu.make_async_copy(
                dst1.at[pl.ds(ck * EB, EB)], dbs[p], sems[2 * p + 1]).wait()

            @plsc.parallel_loop(0, EB // 16, unroll=8)
            def _(i):
                sv = sbs[p][pl.ds(i * 16, 16)]
                dv = dbs[p][pl.ds(i * 16, 16)]
                for f in range(FPW):
                    g = plsc.load_gather(hs[f], [sv])
                    plsc.addupdate_scatter(accs[f], [dv], g)

            @pl.when(ck + 2 < NEB)
            def _():
                pltpu.async_copy(
                    src1.at[pl.ds((ck + 2) * EB, EB)], sbs[p], sems[2 * p])
                pltpu.async_copy(
                    dst1.at[pl.ds((ck + 2) * EB, EB)], dbs[p], sems[2 * p + 1])

        return carry

    lax.fori_loop(0, NEB // 2, ebody, 0)
    for f in range(FPW):
        pltpu.sync_copy(accs[f], outp.at[wid, f])


def _mm_body(x_ref, w_ref, degp_ref, h2s_ref):
    deg = jnp.sum(degp_ref[...][:, 0, :], axis=0)
    dis = lax.rsqrt(jnp.maximum(deg, 1.0))
    h = jnp.dot(x_ref[...], w_ref[...], preferred_element_type=jnp.float32)
    h2 = h * dis[:N, None]
    hT = jnp.concatenate(
        [h2.T, jnp.zeros((D, NP - N), jnp.float32)], axis=1)
    h2s_ref[...] = hT.reshape(NW, FPW, NP)


_mm = pl.pallas_call(
    _mm_body,
    out_shape=jax.ShapeDtypeStruct((NW, FPW, NP), jnp.float32),
)


def _fin_body(outp_ref, degp_ref, b_ref, out_ref):
    deg = jnp.sum(degp_ref[...][:, 0, :], axis=0)
    dis = lax.rsqrt(jnp.maximum(deg, 1.0))
    rows = outp_ref[...].reshape(D, NP)
    out_ref[...] = rows[:, :N].T * dis[:N, None] + b_ref[...]


_fin = pl.pallas_call(
    _fin_body,
    out_shape=jax.ShapeDtypeStruct((N, D), jnp.float32),
)


def kernel(x, edge_index, W, b):
    src1 = edge_index[0]
    dst1 = edge_index[1]
    degp = _deg_kernel(dst1)
    h2s = _mm(x, W, degp)
    outp = _seg_kernel(h2s, src1, dst1)
    return _fin(outp, degp, b.reshape(1, D))
